# Initial kernel scaffold; baseline (speedup 1.0000x reference)
import functools

import jax
import jax.numpy as jnp
from jax import lax
from jax.experimental import pallas as pl
from jax.experimental.pallas import tpu as pltpu
from jax.experimental.pallas import tpu_sc as plsc

N = 50000
E = 800000
D = 64
H = D // 2

N_PAD = 50048
E_PAD = 819200
IDX_ROWS = E_PAD // 128
ROWS_PER_TILE = IDX_ROWS // 16
CHUNKS_PER_TILE = ROWS_PER_TILE // 8
NODE_ROWS_PER_TILE = N_PAD // 16
ZCHUNK = NODE_ROWS_PER_TILE // 4

_SC_MESH = plsc.VectorSubcoreMesh(core_axis_name="c", subcore_axis_name="s")


@functools.partial(
    pl.kernel,
    out_type=[
        jax.ShapeDtypeStruct((N_PAD,), jnp.float32),
        jax.ShapeDtypeStruct((N_PAD,), jnp.float32),
    ],
    mesh=_SC_MESH,
    scratch_types=[
        pltpu.VMEM((8, 128), jnp.int32),
        pltpu.VMEM((128,), jnp.float32),
        pltpu.VMEM((NODE_ROWS_PER_TILE,), jnp.float32),
        pltpu.VMEM_SHARED((N_PAD,), jnp.float32),
    ],
)
def _deg_kernel(src_hbm, dst_hbm, dego_hbm, degi_hbm, idx_v, ones_v, z_v, acc):
    c = lax.axis_index("c")
    s = lax.axis_index("s")

    @pl.loop(0, 8)
    def _fill_ones(i):
        ones_v[pl.ds(i * 16, 16)] = jnp.ones((16,), jnp.float32)

    @pl.loop(0, NODE_ROWS_PER_TILE // 16)
    def _fill_zeros(i):
        z_v[pl.ds(i * 16, 16)] = jnp.zeros((16,), jnp.float32)

    pltpu.sync_copy(z_v, acc.at[pl.ds(s * NODE_ROWS_PER_TILE, NODE_ROWS_PER_TILE)])
    plsc.subcore_barrier()

    @pl.loop(0, CHUNKS_PER_TILE)
    def _edges(g):
        base = s * ROWS_PER_TILE + g * 8

        @pl.when(c == 0)
        def _():
            pltpu.sync_copy(src_hbm.at[pl.ds(base, 8)], idx_v)

        @pl.when(c == 1)
        def _():
            pltpu.sync_copy(dst_hbm.at[pl.ds(base, 8)], idx_v)

        for j in range(8):
            pltpu.sync_copy(ones_v, acc.at[idx_v.at[j]], add=True)

    plsc.subcore_barrier()

    @pl.when(c == 0)
    def _():
        pltpu.sync_copy(
            acc.at[pl.ds(s * NODE_ROWS_PER_TILE, NODE_ROWS_PER_TILE)],
            dego_hbm.at[pl.ds(s * NODE_ROWS_PER_TILE, NODE_ROWS_PER_TILE)],
        )

    @pl.when(c == 1)
    def _():
        pltpu.sync_copy(
            acc.at[pl.ds(s * NODE_ROWS_PER_TILE, NODE_ROWS_PER_TILE)],
            degi_hbm.at[pl.ds(s * NODE_ROWS_PER_TILE, NODE_ROWS_PER_TILE)],
        )


@functools.partial(
    pl.kernel,
    out_type=[
        jax.ShapeDtypeStruct((N_PAD, H), jnp.float32),
        jax.ShapeDtypeStruct((N_PAD, H), jnp.float32),
    ],
    mesh=_SC_MESH,
    scratch_types=[
        pltpu.VMEM((8, 128), jnp.int32),
        pltpu.VMEM((8, 128), jnp.int32),
        pltpu.VMEM((8, 128, H), jnp.float32),
        pltpu.VMEM((ZCHUNK, H), jnp.float32),
        pltpu.VMEM_SHARED((N_PAD, H), jnp.float32),
    ],
)
def _agg_kernel(ylo_hbm, yhi_hbm, src_hbm, dst_hbm, alo_hbm, ahi_hbm,
                sidx, didx, rows, z_v, acc):
    c = lax.axis_index("c")
    s = lax.axis_index("s")

    @pl.loop(0, ZCHUNK)
    def _fill_zeros(i):
        @pl.loop(0, H // 16)
        def _inner(j):
            z_v[i, pl.ds(j * 16, 16)] = jnp.zeros((16,), jnp.float32)

    @pl.loop(0, 4)
    def _zero_acc(k):
        pltpu.sync_copy(
            z_v, acc.at[pl.ds(s * NODE_ROWS_PER_TILE + k * ZCHUNK, ZCHUNK)]
        )

    plsc.subcore_barrier()

    @pl.loop(0, CHUNKS_PER_TILE)
    def _edges(g):
        base = s * ROWS_PER_TILE + g * 8
        pltpu.sync_copy(src_hbm.at[pl.ds(base, 8)], sidx)
        pltpu.sync_copy(dst_hbm.at[pl.ds(base, 8)], didx)
        for j in range(8):
            @pl.when(c == 0)
            def _():
                pltpu.sync_copy(ylo_hbm.at[sidx.at[j]], rows.at[j])

            @pl.when(c == 1)
            def _():
                pltpu.sync_copy(yhi_hbm.at[sidx.at[j]], rows.at[j])

            pltpu.sync_copy(rows.at[j], acc.at[didx.at[j]], add=True)

    plsc.subcore_barrier()

    @pl.when(c == 0)
    def _():
        pltpu.sync_copy(
            acc.at[pl.ds(s * NODE_ROWS_PER_TILE, NODE_ROWS_PER_TILE)],
            alo_hbm.at[pl.ds(s * NODE_ROWS_PER_TILE, NODE_ROWS_PER_TILE)],
        )

    @pl.when(c == 1)
    def _():
        pltpu.sync_copy(
            acc.at[pl.ds(s * NODE_ROWS_PER_TILE, NODE_ROWS_PER_TILE)],
            ahi_hbm.at[pl.ds(s * NODE_ROWS_PER_TILE, NODE_ROWS_PER_TILE)],
        )


_RB = N_PAD // 8


def _tc0_body(x_ref, w_ref, dego_ref, ylo_ref, yhi_ref):
    so = lax.rsqrt(jnp.maximum(dego_ref[...], 1.0))
    y = jnp.dot(x_ref[...], w_ref[...], preferred_element_type=jnp.float32) * so
    ylo_ref[...] = y[:, :H]
    yhi_ref[...] = y[:, H:]


def _tc1_body(alo_ref, ahi_ref, degi_ref, dego_ref, b_ref, w_ref,
              ylo_ref, yhi_ref):
    si = lax.rsqrt(jnp.maximum(degi_ref[...], 1.0))
    so = lax.rsqrt(jnp.maximum(dego_ref[...], 1.0))
    h_lo = jnp.maximum(alo_ref[...] * si + b_ref[:, :H], 0.0)
    h_hi = jnp.maximum(ahi_ref[...] * si + b_ref[:, H:], 0.0)
    y = (
        jnp.dot(h_lo, w_ref[:H, :], preferred_element_type=jnp.float32)
        + jnp.dot(h_hi, w_ref[H:, :], preferred_element_type=jnp.float32)
    ) * so
    ylo_ref[...] = y[:, :H]
    yhi_ref[...] = y[:, H:]


def _tc2_body(alo_ref, ahi_ref, degi_ref, b_ref, out_ref):
    si = lax.rsqrt(jnp.maximum(degi_ref[...], 1.0))
    out_ref[:, :H] = alo_ref[...] * si + b_ref[:, :H]
    out_ref[:, H:] = ahi_ref[...] * si + b_ref[:, H:]


def _row_spec(cols):
    return pl.BlockSpec((_RB, cols), lambda i: (i, 0))


def _full_spec(r, cols):
    return pl.BlockSpec((r, cols), lambda i: (0, 0))


_tc0 = pl.pallas_call(
    _tc0_body,
    grid=(8,),
    in_specs=[_row_spec(D), _full_spec(D, D), _row_spec(1)],
    out_specs=[_row_spec(H), _row_spec(H)],
    out_shape=[
        jax.ShapeDtypeStruct((N_PAD, H), jnp.float32),
        jax.ShapeDtypeStruct((N_PAD, H), jnp.float32),
    ],
)

_tc1 = pl.pallas_call(
    _tc1_body,
    grid=(8,),
    in_specs=[
        _row_spec(H), _row_spec(H), _row_spec(1), _row_spec(1),
        _full_spec(1, D), _full_spec(D, D),
    ],
    out_specs=[_row_spec(H), _row_spec(H)],
    out_shape=[
        jax.ShapeDtypeStruct((N_PAD, H), jnp.float32),
        jax.ShapeDtypeStruct((N_PAD, H), jnp.float32),
    ],
)

_tc2 = pl.pallas_call(
    _tc2_body,
    grid=(8,),
    in_specs=[_row_spec(H), _row_spec(H), _row_spec(1), _full_spec(1, D)],
    out_specs=pl.BlockSpec((_RB, D), lambda i: (i, 0)),
    out_shape=jax.ShapeDtypeStruct((N_PAD, D), jnp.float32),
)


def kernel(edge_index, node_embeddings, W1, b1, W2, b2):
    src = edge_index[0].astype(jnp.int32)
    dst = edge_index[1].astype(jnp.int32)
    pad = jnp.full((E_PAD - E,), N, dtype=jnp.int32)
    srcp = jnp.concatenate([src, pad]).reshape(IDX_ROWS, 128)
    dstp = jnp.concatenate([dst, pad]).reshape(IDX_ROWS, 128)

    x_pad = jnp.zeros((N_PAD, D), jnp.float32).at[:N].set(node_embeddings)

    dego, degi = _deg_kernel(srcp, dstp)
    dego = dego.reshape(N_PAD, 1)
    degi = degi.reshape(N_PAD, 1)

    b1r = b1.reshape(1, D)
    b2r = b2.reshape(1, D)

    ylo, yhi = _tc0(x_pad, W1, dego)
    alo, ahi = _agg_kernel(ylo, yhi, srcp, dstp)
    y2lo, y2hi = _tc1(alo, ahi, degi, dego, b1r, W2)
    a2lo, a2hi = _agg_kernel(y2lo, y2hi, srcp, dstp)
    out = _tc2(a2lo, a2hi, degi, b2r)
    return out[:N]


# speedup vs baseline: 3.8023x; 3.8023x over previous
import functools

import jax
import jax.numpy as jnp
from jax import lax
from jax.experimental import pallas as pl
from jax.experimental.pallas import tpu as pltpu
from jax.experimental.pallas import tpu_sc as plsc

N = 50000
E = 800000
D = 64
H = D // 2

N_PAD = 50176
E_PAD = 819200
IDX_ROWS = E_PAD // 128
ROWS_PER_TILE = IDX_ROWS // 16
CHUNKS_PER_TILE = ROWS_PER_TILE // 8
NODE_ROWS_PER_TILE = N_PAD // 16
ZCHUNK = NODE_ROWS_PER_TILE // 8
NZCOPY = NODE_ROWS_PER_TILE // ZCHUNK

_SC_MESH = plsc.VectorSubcoreMesh(core_axis_name="c", subcore_axis_name="s")


@functools.partial(
    pl.kernel,
    out_type=[
        jax.ShapeDtypeStruct((N_PAD,), jnp.float32),
        jax.ShapeDtypeStruct((N_PAD,), jnp.float32),
    ],
    mesh=_SC_MESH,
    compiler_params=pltpu.CompilerParams(use_tc_tiling_on_sc=False),
    scratch_types=[
        pltpu.VMEM((8, 128), jnp.int32),
        pltpu.VMEM((128,), jnp.float32),
        pltpu.VMEM((NODE_ROWS_PER_TILE,), jnp.float32),
        pltpu.VMEM_SHARED((N_PAD,), jnp.float32),
    ],
)
def _deg_kernel(src_hbm, dst_hbm, dego_hbm, degi_hbm, idx_v, ones_v, z_v, acc):
    c = lax.axis_index("c")
    s = lax.axis_index("s")

    @pl.loop(0, 8)
    def _fill_ones(i):
        ones_v[pl.ds(i * 16, 16)] = jnp.ones((16,), jnp.float32)

    @pl.loop(0, NODE_ROWS_PER_TILE // 16)
    def _fill_zeros(i):
        z_v[pl.ds(i * 16, 16)] = jnp.zeros((16,), jnp.float32)

    pltpu.sync_copy(z_v, acc.at[pl.ds(s * NODE_ROWS_PER_TILE, NODE_ROWS_PER_TILE)])
    plsc.subcore_barrier()

    @pl.loop(0, CHUNKS_PER_TILE)
    def _edges(g):
        base = s * ROWS_PER_TILE + g * 8

        @pl.when(c == 0)
        def _():
            pltpu.sync_copy(src_hbm.at[pl.ds(base, 8)], idx_v)

        @pl.when(c == 1)
        def _():
            pltpu.sync_copy(dst_hbm.at[pl.ds(base, 8)], idx_v)

        for j in range(8):
            pltpu.sync_copy(ones_v, acc.at[idx_v.at[j]], add=True)

    plsc.subcore_barrier()

    pltpu.sync_copy(acc.at[pl.ds(s * NODE_ROWS_PER_TILE, NODE_ROWS_PER_TILE)], z_v)

    @pl.when(c == 0)
    def _():
        pltpu.sync_copy(
            z_v, dego_hbm.at[pl.ds(s * NODE_ROWS_PER_TILE, NODE_ROWS_PER_TILE)]
        )

    @pl.when(c == 1)
    def _():
        pltpu.sync_copy(
            z_v, degi_hbm.at[pl.ds(s * NODE_ROWS_PER_TILE, NODE_ROWS_PER_TILE)]
        )


@functools.partial(
    pl.kernel,
    out_type=[
        jax.ShapeDtypeStruct((N_PAD, H), jnp.float32),
        jax.ShapeDtypeStruct((N_PAD, H), jnp.float32),
    ],
    mesh=_SC_MESH,
    compiler_params=pltpu.CompilerParams(use_tc_tiling_on_sc=False),
    scratch_types=[
        pltpu.VMEM((8, 128), jnp.int32),
        pltpu.VMEM((8, 128), jnp.int32),
        pltpu.VMEM((2, 128, H), jnp.float32),
        pltpu.VMEM((ZCHUNK, H), jnp.float32),
        pltpu.VMEM_SHARED((N_PAD, H), jnp.float32),
    ],
)
def _agg_kernel(ylo_hbm, yhi_hbm, src_hbm, dst_hbm, alo_hbm, ahi_hbm,
                sidx, didx, rows, z_v, acc):
    c = lax.axis_index("c")
    s = lax.axis_index("s")

    @pl.loop(0, ZCHUNK)
    def _fill_zeros(i):
        @pl.loop(0, H // 16)
        def _inner(j):
            z_v[i, pl.ds(j * 16, 16)] = jnp.zeros((16,), jnp.float32)

    @pl.loop(0, NZCOPY)
    def _zero_acc(k):
        pltpu.sync_copy(
            z_v, acc.at[pl.ds(s * NODE_ROWS_PER_TILE + k * ZCHUNK, ZCHUNK)]
        )

    plsc.subcore_barrier()

    @pl.loop(0, CHUNKS_PER_TILE)
    def _edges(g):
        base = s * ROWS_PER_TILE + g * 8
        pltpu.sync_copy(src_hbm.at[pl.ds(base, 8)], sidx)
        pltpu.sync_copy(dst_hbm.at[pl.ds(base, 8)], didx)
        for j in range(8):
            @pl.when(c == 0)
            def _():
                pltpu.sync_copy(ylo_hbm.at[sidx.at[j]], rows.at[j % 2])

            @pl.when(c == 1)
            def _():
                pltpu.sync_copy(yhi_hbm.at[sidx.at[j]], rows.at[j % 2])

            pltpu.sync_copy(rows.at[j % 2], acc.at[didx.at[j]], add=True)

    plsc.subcore_barrier()

    @pl.loop(0, NZCOPY)
    def _copy_out(k):
        chunk = pl.ds(s * NODE_ROWS_PER_TILE + k * ZCHUNK, ZCHUNK)
        pltpu.sync_copy(acc.at[chunk], z_v)

        @pl.when(c == 0)
        def _():
            pltpu.sync_copy(z_v, alo_hbm.at[chunk])

        @pl.when(c == 1)
        def _():
            pltpu.sync_copy(z_v, ahi_hbm.at[chunk])


_RB = N_PAD // 8


def _tc0_body(x_ref, w_ref, dego_ref, ylo_ref, yhi_ref):
    so = lax.rsqrt(jnp.maximum(dego_ref[...], 1.0))
    y = jnp.dot(x_ref[...], w_ref[...], preferred_element_type=jnp.float32) * so
    ylo_ref[...] = y[:, :H]
    yhi_ref[...] = y[:, H:]


def _tc1_body(alo_ref, ahi_ref, degi_ref, dego_ref, b_ref, w_ref,
              ylo_ref, yhi_ref):
    si = lax.rsqrt(jnp.maximum(degi_ref[...], 1.0))
    so = lax.rsqrt(jnp.maximum(dego_ref[...], 1.0))
    h_lo = jnp.maximum(alo_ref[...] * si + b_ref[:, :H], 0.0)
    h_hi = jnp.maximum(ahi_ref[...] * si + b_ref[:, H:], 0.0)
    y = (
        jnp.dot(h_lo, w_ref[:H, :], preferred_element_type=jnp.float32)
        + jnp.dot(h_hi, w_ref[H:, :], preferred_element_type=jnp.float32)
    ) * so
    ylo_ref[...] = y[:, :H]
    yhi_ref[...] = y[:, H:]


def _tc2_body(alo_ref, ahi_ref, degi_ref, b_ref, out_ref):
    si = lax.rsqrt(jnp.maximum(degi_ref[...], 1.0))
    out_ref[:, :H] = alo_ref[...] * si + b_ref[:, :H]
    out_ref[:, H:] = ahi_ref[...] * si + b_ref[:, H:]


def _row_spec(cols):
    return pl.BlockSpec((_RB, cols), lambda i: (i, 0))


def _full_spec(r, cols):
    return pl.BlockSpec((r, cols), lambda i: (0, 0))


_tc0 = pl.pallas_call(
    _tc0_body,
    grid=(8,),
    in_specs=[_row_spec(D), _full_spec(D, D), _row_spec(1)],
    out_specs=[_row_spec(H), _row_spec(H)],
    out_shape=[
        jax.ShapeDtypeStruct((N_PAD, H), jnp.float32),
        jax.ShapeDtypeStruct((N_PAD, H), jnp.float32),
    ],
)

_tc1 = pl.pallas_call(
    _tc1_body,
    grid=(8,),
    in_specs=[
        _row_spec(H), _row_spec(H), _row_spec(1), _row_spec(1),
        _full_spec(1, D), _full_spec(D, D),
    ],
    out_specs=[_row_spec(H), _row_spec(H)],
    out_shape=[
        jax.ShapeDtypeStruct((N_PAD, H), jnp.float32),
        jax.ShapeDtypeStruct((N_PAD, H), jnp.float32),
    ],
)

_tc2 = pl.pallas_call(
    _tc2_body,
    grid=(8,),
    in_specs=[_row_spec(H), _row_spec(H), _row_spec(1), _full_spec(1, D)],
    out_specs=pl.BlockSpec((_RB, D), lambda i: (i, 0)),
    out_shape=jax.ShapeDtypeStruct((N_PAD, D), jnp.float32),
)


def kernel(edge_index, node_embeddings, W1, b1, W2, b2):
    src = edge_index[0].astype(jnp.int32)
    dst = edge_index[1].astype(jnp.int32)
    pad = jnp.full((E_PAD - E,), N, dtype=jnp.int32)
    srcp = jnp.concatenate([src, pad]).reshape(IDX_ROWS, 128)
    dstp = jnp.concatenate([dst, pad]).reshape(IDX_ROWS, 128)

    x_pad = jnp.zeros((N_PAD, D), jnp.float32).at[:N].set(node_embeddings)

    dego, degi = _deg_kernel(srcp, dstp)
    dego = dego.reshape(N_PAD, 1)
    degi = degi.reshape(N_PAD, 1)

    b1r = b1.reshape(1, D)
    b2r = b2.reshape(1, D)

    ylo, yhi = _tc0(x_pad, W1, dego)
    alo, ahi = _agg_kernel(ylo, yhi, srcp, dstp)
    y2lo, y2hi = _tc1(alo, ahi, degi, dego, b1r, W2)
    a2lo, a2hi = _agg_kernel(y2lo, y2hi, srcp, dstp)
    out = _tc2(a2lo, a2hi, degi, b2r)
    return out[:N]


# speedup vs baseline: 4.5567x; 1.1984x over previous
import functools

import jax
import jax.numpy as jnp
from jax import lax
from jax.experimental import pallas as pl
from jax.experimental.pallas import tpu as pltpu
from jax.experimental.pallas import tpu_sc as plsc

N = 50000
E = 800000
D = 64
H = D // 2

N_PAD = 50176
E_PAD = 819200
IDX_ROWS = E_PAD // 128
ROWS_PER_TILE = IDX_ROWS // 16
CHUNKS_PER_TILE = ROWS_PER_TILE // 8
NODE_ROWS_PER_TILE = N_PAD // 16
ZCHUNK = NODE_ROWS_PER_TILE // 28
NZCOPY = NODE_ROWS_PER_TILE // ZCHUNK

_SC_MESH = plsc.VectorSubcoreMesh(core_axis_name="c", subcore_axis_name="s")


@functools.partial(
    pl.kernel,
    out_type=[
        jax.ShapeDtypeStruct((N_PAD,), jnp.float32),
        jax.ShapeDtypeStruct((N_PAD,), jnp.float32),
    ],
    mesh=_SC_MESH,
    compiler_params=pltpu.CompilerParams(use_tc_tiling_on_sc=False),
    scratch_types=[
        pltpu.VMEM((8, 128), jnp.int32),
        pltpu.VMEM((128,), jnp.float32),
        pltpu.VMEM((NODE_ROWS_PER_TILE,), jnp.float32),
        pltpu.VMEM_SHARED((N_PAD,), jnp.float32),
    ],
)
def _deg_kernel(src_hbm, dst_hbm, dego_hbm, degi_hbm, idx_v, ones_v, z_v, acc):
    c = lax.axis_index("c")
    s = lax.axis_index("s")

    @pl.loop(0, 8)
    def _fill_ones(i):
        ones_v[pl.ds(i * 16, 16)] = jnp.ones((16,), jnp.float32)

    @pl.loop(0, NODE_ROWS_PER_TILE // 16)
    def _fill_zeros(i):
        z_v[pl.ds(i * 16, 16)] = jnp.zeros((16,), jnp.float32)

    pltpu.sync_copy(z_v, acc.at[pl.ds(s * NODE_ROWS_PER_TILE, NODE_ROWS_PER_TILE)])
    plsc.subcore_barrier()

    @pl.loop(0, CHUNKS_PER_TILE)
    def _edges(g):
        base = s * ROWS_PER_TILE + g * 8

        @pl.when(c == 0)
        def _():
            pltpu.sync_copy(src_hbm.at[pl.ds(base, 8)], idx_v)

        @pl.when(c == 1)
        def _():
            pltpu.sync_copy(dst_hbm.at[pl.ds(base, 8)], idx_v)

        for j in range(8):
            pltpu.sync_copy(ones_v, acc.at[idx_v.at[j]], add=True)

    plsc.subcore_barrier()

    pltpu.sync_copy(acc.at[pl.ds(s * NODE_ROWS_PER_TILE, NODE_ROWS_PER_TILE)], z_v)

    @pl.when(c == 0)
    def _():
        pltpu.sync_copy(
            z_v, dego_hbm.at[pl.ds(s * NODE_ROWS_PER_TILE, NODE_ROWS_PER_TILE)]
        )

    @pl.when(c == 1)
    def _():
        pltpu.sync_copy(
            z_v, degi_hbm.at[pl.ds(s * NODE_ROWS_PER_TILE, NODE_ROWS_PER_TILE)]
        )


@functools.partial(
    pl.kernel,
    out_type=[
        jax.ShapeDtypeStruct((N_PAD, H), jnp.float32),
        jax.ShapeDtypeStruct((N_PAD, H), jnp.float32),
    ],
    mesh=_SC_MESH,
    compiler_params=pltpu.CompilerParams(use_tc_tiling_on_sc=False),
    scratch_types=[
        pltpu.VMEM((3, 8, 128), jnp.int32),
        pltpu.VMEM((3, 8, 128), jnp.int32),
        pltpu.VMEM((2, 128, H), jnp.float32),
        pltpu.VMEM((ZCHUNK, H), jnp.float32),
        pltpu.VMEM_SHARED((N_PAD, H), jnp.float32),
        pltpu.SemaphoreType.DMA,
        pltpu.SemaphoreType.DMA,
        pltpu.SemaphoreType.DMA,
    ],
)
def _agg_kernel(ylo_hbm, yhi_hbm, src_hbm, dst_hbm, alo_hbm, ahi_hbm,
                sidx, didx, rows, z_v, acc, isem, gsem, ssem):
    c = lax.axis_index("c")
    s = lax.axis_index("s")
    n_rows = ROWS_PER_TILE

    @pl.loop(0, ZCHUNK)
    def _fill_zeros(i):
        @pl.loop(0, H // 16)
        def _inner(j):
            z_v[i, pl.ds(j * 16, 16)] = jnp.zeros((16,), jnp.float32)

    @pl.loop(0, NZCOPY)
    def _zero_acc(k):
        pltpu.sync_copy(
            z_v, acc.at[pl.ds(s * NODE_ROWS_PER_TILE + k * ZCHUNK, ZCHUNK)]
        )

    plsc.subcore_barrier()

    def _fire_gather(slot, j, buf):
        @pl.when(c == 0)
        def _():
            pltpu.async_copy(ylo_hbm.at[sidx.at[slot, j]], rows.at[buf], gsem)

        @pl.when(c == 1)
        def _():
            pltpu.async_copy(yhi_hbm.at[sidx.at[slot, j]], rows.at[buf], gsem)

    def _wait_gather():
        pltpu.make_async_copy(ylo_hbm.at[sidx.at[0, 0]], rows.at[0], gsem).wait()

    def _wait_scatter():
        pltpu.make_async_copy(rows.at[0], acc.at[didx.at[0, 0]], ssem).wait()

    def _wait_iload():
        pltpu.make_async_copy(src_hbm.at[pl.ds(0, 8)], sidx.at[0], isem).wait()
        pltpu.make_async_copy(dst_hbm.at[pl.ds(0, 8)], didx.at[0], isem).wait()

    pltpu.sync_copy(src_hbm.at[pl.ds(s * n_rows, 8)], sidx.at[0])
    pltpu.sync_copy(dst_hbm.at[pl.ds(s * n_rows, 8)], didx.at[0])
    _fire_gather(0, 0, 0)

    @pl.loop(0, n_rows)
    def _edges(r):
        g = r // 8
        j = r % 8
        b = r % 2
        slot = g % 3

        @pl.when(j == 0)
        def _():
            @pl.when(g > 0)
            def _():
                _wait_iload()

            @pl.when(g < CHUNKS_PER_TILE - 1)
            def _():
                nslot = (g + 1) % 3
                base = s * n_rows + (g + 1) * 8
                pltpu.async_copy(src_hbm.at[pl.ds(base, 8)], sidx.at[nslot], isem)
                pltpu.async_copy(dst_hbm.at[pl.ds(base, 8)], didx.at[nslot], isem)

        _wait_gather()

        @pl.when(r > 0)
        def _():
            _wait_scatter()

        @pl.when(r < n_rows - 1)
        def _():
            r1 = r + 1
            _fire_gather((r1 // 8) % 3, r1 % 8, r1 % 2)

        pltpu.async_copy(rows.at[b], acc.at[didx.at[slot, j]], ssem, add=True)

    _wait_scatter()
    plsc.subcore_barrier()

    @pl.loop(0, NZCOPY)
    def _copy_out(k):
        chunk = pl.ds(s * NODE_ROWS_PER_TILE + k * ZCHUNK, ZCHUNK)
        pltpu.sync_copy(acc.at[chunk], z_v)

        @pl.when(c == 0)
        def _():
            pltpu.sync_copy(z_v, alo_hbm.at[chunk])

        @pl.when(c == 1)
        def _():
            pltpu.sync_copy(z_v, ahi_hbm.at[chunk])


_RB = N_PAD // 8


def _tc0_body(x_ref, w_ref, dego_ref, ylo_ref, yhi_ref):
    so = lax.rsqrt(jnp.maximum(dego_ref[...], 1.0))
    y = jnp.dot(x_ref[...], w_ref[...], preferred_element_type=jnp.float32) * so
    ylo_ref[...] = y[:, :H]
    yhi_ref[...] = y[:, H:]


def _tc1_body(alo_ref, ahi_ref, degi_ref, dego_ref, b_ref, w_ref,
              ylo_ref, yhi_ref):
    si = lax.rsqrt(jnp.maximum(degi_ref[...], 1.0))
    so = lax.rsqrt(jnp.maximum(dego_ref[...], 1.0))
    h_lo = jnp.maximum(alo_ref[...] * si + b_ref[:, :H], 0.0)
    h_hi = jnp.maximum(ahi_ref[...] * si + b_ref[:, H:], 0.0)
    y = (
        jnp.dot(h_lo, w_ref[:H, :], preferred_element_type=jnp.float32)
        + jnp.dot(h_hi, w_ref[H:, :], preferred_element_type=jnp.float32)
    ) * so
    ylo_ref[...] = y[:, :H]
    yhi_ref[...] = y[:, H:]


def _tc2_body(alo_ref, ahi_ref, degi_ref, b_ref, out_ref):
    si = lax.rsqrt(jnp.maximum(degi_ref[...], 1.0))
    out_ref[:, :H] = alo_ref[...] * si + b_ref[:, :H]
    out_ref[:, H:] = ahi_ref[...] * si + b_ref[:, H:]


def _row_spec(cols):
    return pl.BlockSpec((_RB, cols), lambda i: (i, 0))


def _full_spec(r, cols):
    return pl.BlockSpec((r, cols), lambda i: (0, 0))


_tc0 = pl.pallas_call(
    _tc0_body,
    grid=(8,),
    in_specs=[_row_spec(D), _full_spec(D, D), _row_spec(1)],
    out_specs=[_row_spec(H), _row_spec(H)],
    out_shape=[
        jax.ShapeDtypeStruct((N_PAD, H), jnp.float32),
        jax.ShapeDtypeStruct((N_PAD, H), jnp.float32),
    ],
)

_tc1 = pl.pallas_call(
    _tc1_body,
    grid=(8,),
    in_specs=[
        _row_spec(H), _row_spec(H), _row_spec(1), _row_spec(1),
        _full_spec(1, D), _full_spec(D, D),
    ],
    out_specs=[_row_spec(H), _row_spec(H)],
    out_shape=[
        jax.ShapeDtypeStruct((N_PAD, H), jnp.float32),
        jax.ShapeDtypeStruct((N_PAD, H), jnp.float32),
    ],
)

_tc2 = pl.pallas_call(
    _tc2_body,
    grid=(8,),
    in_specs=[_row_spec(H), _row_spec(H), _row_spec(1), _full_spec(1, D)],
    out_specs=pl.BlockSpec((_RB, D), lambda i: (i, 0)),
    out_shape=jax.ShapeDtypeStruct((N_PAD, D), jnp.float32),
)


def kernel(edge_index, node_embeddings, W1, b1, W2, b2):
    src = edge_index[0].astype(jnp.int32)
    dst = edge_index[1].astype(jnp.int32)
    pad = jnp.full((E_PAD - E,), N, dtype=jnp.int32)
    srcp = jnp.concatenate([src, pad]).reshape(IDX_ROWS, 128)
    dstp = jnp.concatenate([dst, pad]).reshape(IDX_ROWS, 128)

    x_pad = jnp.zeros((N_PAD, D), jnp.float32).at[:N].set(node_embeddings)

    dego, degi = _deg_kernel(srcp, dstp)
    dego = dego.reshape(N_PAD, 1)
    degi = degi.reshape(N_PAD, 1)

    b1r = b1.reshape(1, D)
    b2r = b2.reshape(1, D)

    ylo, yhi = _tc0(x_pad, W1, dego)
    alo, ahi = _agg_kernel(ylo, yhi, srcp, dstp)
    y2lo, y2hi = _tc1(alo, ahi, degi, dego, b1r, W2)
    a2lo, a2hi = _agg_kernel(y2lo, y2hi, srcp, dstp)
    out = _tc2(a2lo, a2hi, degi, b2r)
    return out[:N]


# speedup vs baseline: 5.6424x; 1.2383x over previous
import functools

import jax
import jax.numpy as jnp
from jax import lax
from jax.experimental import pallas as pl
from jax.experimental.pallas import tpu as pltpu
from jax.experimental.pallas import tpu_sc as plsc

N = 50000
E = 800000
D = 64
H = D // 2

N_PAD = 50176
E_PAD = 819200
IDX_ROWS = E_PAD // 128
ROWS_PER_TILE = IDX_ROWS // 16
CHUNKS_PER_TILE = ROWS_PER_TILE // 8
NODE_ROWS_PER_TILE = N_PAD // 16
ZCHUNK = NODE_ROWS_PER_TILE // 28
NZCOPY = NODE_ROWS_PER_TILE // ZCHUNK

_SC_MESH = plsc.VectorSubcoreMesh(core_axis_name="c", subcore_axis_name="s")


@functools.partial(
    pl.kernel,
    out_type=[
        jax.ShapeDtypeStruct((N_PAD,), jnp.float32),
        jax.ShapeDtypeStruct((N_PAD,), jnp.float32),
    ],
    mesh=_SC_MESH,
    compiler_params=pltpu.CompilerParams(use_tc_tiling_on_sc=False),
    scratch_types=[
        pltpu.VMEM((8, 128), jnp.int32),
        pltpu.VMEM((128,), jnp.float32),
        pltpu.VMEM((NODE_ROWS_PER_TILE,), jnp.float32),
        pltpu.VMEM_SHARED((N_PAD,), jnp.float32),
    ],
)
def _deg_kernel(src_hbm, dst_hbm, dego_hbm, degi_hbm, idx_v, ones_v, z_v, acc):
    c = lax.axis_index("c")
    s = lax.axis_index("s")

    @pl.loop(0, 8)
    def _fill_ones(i):
        ones_v[pl.ds(i * 16, 16)] = jnp.ones((16,), jnp.float32)

    @pl.loop(0, NODE_ROWS_PER_TILE // 16)
    def _fill_zeros(i):
        z_v[pl.ds(i * 16, 16)] = jnp.zeros((16,), jnp.float32)

    pltpu.sync_copy(z_v, acc.at[pl.ds(s * NODE_ROWS_PER_TILE, NODE_ROWS_PER_TILE)])
    plsc.subcore_barrier()

    @pl.loop(0, CHUNKS_PER_TILE)
    def _edges(g):
        base = s * ROWS_PER_TILE + g * 8

        @pl.when(c == 0)
        def _():
            pltpu.sync_copy(src_hbm.at[pl.ds(base, 8)], idx_v)

        @pl.when(c == 1)
        def _():
            pltpu.sync_copy(dst_hbm.at[pl.ds(base, 8)], idx_v)

        for j in range(8):
            pltpu.sync_copy(ones_v, acc.at[idx_v.at[j]], add=True)

    plsc.subcore_barrier()

    pltpu.sync_copy(acc.at[pl.ds(s * NODE_ROWS_PER_TILE, NODE_ROWS_PER_TILE)], z_v)

    @pl.when(c == 0)
    def _():
        pltpu.sync_copy(
            z_v, dego_hbm.at[pl.ds(s * NODE_ROWS_PER_TILE, NODE_ROWS_PER_TILE)]
        )

    @pl.when(c == 1)
    def _():
        pltpu.sync_copy(
            z_v, degi_hbm.at[pl.ds(s * NODE_ROWS_PER_TILE, NODE_ROWS_PER_TILE)]
        )


@functools.partial(
    pl.kernel,
    out_type=[
        jax.ShapeDtypeStruct((N_PAD, H), jnp.float32),
        jax.ShapeDtypeStruct((N_PAD, H), jnp.float32),
    ],
    mesh=_SC_MESH,
    compiler_params=pltpu.CompilerParams(use_tc_tiling_on_sc=False),
    scratch_types=[
        pltpu.VMEM((3, 8, 128), jnp.int32),
        pltpu.VMEM((3, 8, 128), jnp.int32),
        pltpu.VMEM((4, 128, H), jnp.float32),
        pltpu.VMEM((ZCHUNK, H), jnp.float32),
        pltpu.VMEM_SHARED((N_PAD, H), jnp.float32),
        pltpu.SemaphoreType.DMA,
        pltpu.SemaphoreType.DMA,
        pltpu.SemaphoreType.DMA,
        pltpu.SemaphoreType.DMA,
        pltpu.SemaphoreType.DMA,
    ],
)
def _agg_kernel(ylo_hbm, yhi_hbm, src_hbm, dst_hbm, alo_hbm, ahi_hbm,
                sidx, didx, rows, z_v, acc, isem, gsem0, gsem1, ssem0, ssem1):
    c = lax.axis_index("c")
    s = lax.axis_index("s")
    n_rows = ROWS_PER_TILE

    @pl.loop(0, ZCHUNK)
    def _fill_zeros(i):
        @pl.loop(0, H // 16)
        def _inner(j):
            z_v[i, pl.ds(j * 16, 16)] = jnp.zeros((16,), jnp.float32)

    @pl.loop(0, NZCOPY)
    def _zero_acc(k):
        pltpu.sync_copy(
            z_v, acc.at[pl.ds(s * NODE_ROWS_PER_TILE + k * ZCHUNK, ZCHUNK)]
        )

    plsc.subcore_barrier()

    def _fire_gather(r):
        slot = (r // 8) % 3
        j = r % 8
        buf = r % 4

        def _do(sem):
            @pl.when(c == 0)
            def _():
                pltpu.async_copy(ylo_hbm.at[sidx.at[slot, j]], rows.at[buf], sem)

            @pl.when(c == 1)
            def _():
                pltpu.async_copy(yhi_hbm.at[sidx.at[slot, j]], rows.at[buf], sem)

        @pl.when(r % 2 == 0)
        def _():
            _do(gsem0)

        @pl.when(r % 2 == 1)
        def _():
            _do(gsem1)

    def _wait_gather(r):
        @pl.when(r % 2 == 0)
        def _():
            pltpu.make_async_copy(ylo_hbm.at[sidx.at[0, 0]], rows.at[0], gsem0).wait()

        @pl.when(r % 2 == 1)
        def _():
            pltpu.make_async_copy(ylo_hbm.at[sidx.at[0, 0]], rows.at[0], gsem1).wait()

    def _fire_scatter(r):
        slot = (r // 8) % 3
        j = r % 8
        buf = r % 4

        @pl.when(r % 2 == 0)
        def _():
            pltpu.async_copy(rows.at[buf], acc.at[didx.at[slot, j]], ssem0, add=True)

        @pl.when(r % 2 == 1)
        def _():
            pltpu.async_copy(rows.at[buf], acc.at[didx.at[slot, j]], ssem1, add=True)

    def _wait_scatter(r):
        @pl.when(r % 2 == 0)
        def _():
            pltpu.make_async_copy(rows.at[0], acc.at[didx.at[0, 0]], ssem0).wait()

        @pl.when(r % 2 == 1)
        def _():
            pltpu.make_async_copy(rows.at[0], acc.at[didx.at[0, 0]], ssem1).wait()

    def _wait_iload():
        pltpu.make_async_copy(src_hbm.at[pl.ds(0, 8)], sidx.at[0], isem).wait()
        pltpu.make_async_copy(dst_hbm.at[pl.ds(0, 8)], didx.at[0], isem).wait()

    pltpu.sync_copy(src_hbm.at[pl.ds(s * n_rows, 8)], sidx.at[0])
    pltpu.sync_copy(dst_hbm.at[pl.ds(s * n_rows, 8)], didx.at[0])
    _fire_gather(0)
    _fire_gather(1)

    @pl.loop(0, n_rows)
    def _edges(r):
        g = r // 8
        j = r % 8

        @pl.when(jnp.logical_and(j == 0, g < CHUNKS_PER_TILE - 1))
        def _():
            nslot = (g + 1) % 3
            base = s * n_rows + (g + 1) * 8
            pltpu.async_copy(src_hbm.at[pl.ds(base, 8)], sidx.at[nslot], isem)
            pltpu.async_copy(dst_hbm.at[pl.ds(base, 8)], didx.at[nslot], isem)

        @pl.when(jnp.logical_and(j == 6, g < CHUNKS_PER_TILE - 1))
        def _():
            _wait_iload()

        _wait_gather(r)

        @pl.when(r >= 2)
        def _():
            _wait_scatter(r)

        @pl.when(r < n_rows - 2)
        def _():
            _fire_gather(r + 2)

        _fire_scatter(r)

    _wait_scatter(0)
    _wait_scatter(1)
    plsc.subcore_barrier()

    @pl.loop(0, NZCOPY)
    def _copy_out(k):
        chunk = pl.ds(s * NODE_ROWS_PER_TILE + k * ZCHUNK, ZCHUNK)
        pltpu.sync_copy(acc.at[chunk], z_v)

        @pl.when(c == 0)
        def _():
            pltpu.sync_copy(z_v, alo_hbm.at[chunk])

        @pl.when(c == 1)
        def _():
            pltpu.sync_copy(z_v, ahi_hbm.at[chunk])


_RB = N_PAD // 8


def _tc0_body(x_ref, w_ref, dego_ref, ylo_ref, yhi_ref):
    so = lax.rsqrt(jnp.maximum(dego_ref[...], 1.0))
    y = jnp.dot(x_ref[...], w_ref[...], preferred_element_type=jnp.float32) * so
    ylo_ref[...] = y[:, :H]
    yhi_ref[...] = y[:, H:]


def _tc1_body(alo_ref, ahi_ref, degi_ref, dego_ref, b_ref, w_ref,
              ylo_ref, yhi_ref):
    si = lax.rsqrt(jnp.maximum(degi_ref[...], 1.0))
    so = lax.rsqrt(jnp.maximum(dego_ref[...], 1.0))
    h_lo = jnp.maximum(alo_ref[...] * si + b_ref[:, :H], 0.0)
    h_hi = jnp.maximum(ahi_ref[...] * si + b_ref[:, H:], 0.0)
    y = (
        jnp.dot(h_lo, w_ref[:H, :], preferred_element_type=jnp.float32)
        + jnp.dot(h_hi, w_ref[H:, :], preferred_element_type=jnp.float32)
    ) * so
    ylo_ref[...] = y[:, :H]
    yhi_ref[...] = y[:, H:]


def _tc2_body(alo_ref, ahi_ref, degi_ref, b_ref, out_ref):
    si = lax.rsqrt(jnp.maximum(degi_ref[...], 1.0))
    out_ref[:, :H] = alo_ref[...] * si + b_ref[:, :H]
    out_ref[:, H:] = ahi_ref[...] * si + b_ref[:, H:]


def _row_spec(cols):
    return pl.BlockSpec((_RB, cols), lambda i: (i, 0))


def _full_spec(r, cols):
    return pl.BlockSpec((r, cols), lambda i: (0, 0))


_tc0 = pl.pallas_call(
    _tc0_body,
    grid=(8,),
    in_specs=[_row_spec(D), _full_spec(D, D), _row_spec(1)],
    out_specs=[_row_spec(H), _row_spec(H)],
    out_shape=[
        jax.ShapeDtypeStruct((N_PAD, H), jnp.float32),
        jax.ShapeDtypeStruct((N_PAD, H), jnp.float32),
    ],
)

_tc1 = pl.pallas_call(
    _tc1_body,
    grid=(8,),
    in_specs=[
        _row_spec(H), _row_spec(H), _row_spec(1), _row_spec(1),
        _full_spec(1, D), _full_spec(D, D),
    ],
    out_specs=[_row_spec(H), _row_spec(H)],
    out_shape=[
        jax.ShapeDtypeStruct((N_PAD, H), jnp.float32),
        jax.ShapeDtypeStruct((N_PAD, H), jnp.float32),
    ],
)

_tc2 = pl.pallas_call(
    _tc2_body,
    grid=(8,),
    in_specs=[_row_spec(H), _row_spec(H), _row_spec(1), _full_spec(1, D)],
    out_specs=pl.BlockSpec((_RB, D), lambda i: (i, 0)),
    out_shape=jax.ShapeDtypeStruct((N_PAD, D), jnp.float32),
)


def kernel(edge_index, node_embeddings, W1, b1, W2, b2):
    src = edge_index[0].astype(jnp.int32)
    dst = edge_index[1].astype(jnp.int32)
    pad = jnp.full((E_PAD - E,), N, dtype=jnp.int32)
    srcp = jnp.concatenate([src, pad]).reshape(IDX_ROWS, 128)
    dstp = jnp.concatenate([dst, pad]).reshape(IDX_ROWS, 128)

    x_pad = jnp.zeros((N_PAD, D), jnp.float32).at[:N].set(node_embeddings)

    dego, degi = _deg_kernel(srcp, dstp)
    dego = dego.reshape(N_PAD, 1)
    degi = degi.reshape(N_PAD, 1)

    b1r = b1.reshape(1, D)
    b2r = b2.reshape(1, D)

    ylo, yhi = _tc0(x_pad, W1, dego)
    alo, ahi = _agg_kernel(ylo, yhi, srcp, dstp)
    y2lo, y2hi = _tc1(alo, ahi, degi, dego, b1r, W2)
    a2lo, a2hi = _agg_kernel(y2lo, y2hi, srcp, dstp)
    out = _tc2(a2lo, a2hi, degi, b2r)
    return out[:N]


# speedup vs baseline: 9.1828x; 1.6275x over previous
import functools

import jax
import jax.numpy as jnp
from jax import lax
from jax.experimental import pallas as pl
from jax.experimental.pallas import tpu as pltpu
from jax.experimental.pallas import tpu_sc as plsc

N = 50000
E = 800000
D = 64
H = D // 2

N_PAD = 50176
E_PAD = 802816
N_EXTRA = E_PAD - E
IDX_ROWS = E_PAD // 128
ROWS_PER_TILE = IDX_ROWS // 16
NODE_ROWS_PER_TILE = N_PAD // 16

AGG_ICHUNK = 4
AGG_NCHUNK = ROWS_PER_TILE // AGG_ICHUNK
ZROWS = 56
NZCOPY = NODE_ROWS_PER_TILE // ZROWS
CORORS = 112
NCOCOPY = NODE_ROWS_PER_TILE // CORORS

DEG_ICHUNK = 8
DEG_NCHUNK = ROWS_PER_TILE // DEG_ICHUNK

_SC_MESH = plsc.VectorSubcoreMesh(core_axis_name="c", subcore_axis_name="s")


@functools.partial(
    pl.kernel,
    out_type=[
        jax.ShapeDtypeStruct((N_PAD,), jnp.float32),
        jax.ShapeDtypeStruct((N_PAD,), jnp.float32),
    ],
    mesh=_SC_MESH,
    compiler_params=pltpu.CompilerParams(use_tc_tiling_on_sc=False),
    scratch_types=[
        pltpu.VMEM((3, DEG_ICHUNK, 128), jnp.int32),
        pltpu.VMEM((128,), jnp.float32),
        pltpu.VMEM((NODE_ROWS_PER_TILE,), jnp.float32),
        pltpu.VMEM_SHARED((N_PAD,), jnp.float32),
        pltpu.SemaphoreType.DMA,
        pltpu.SemaphoreType.DMA,
        pltpu.SemaphoreType.DMA,
        pltpu.SemaphoreType.DMA,
    ],
)
def _deg_kernel(src_hbm, dst_hbm, dego_hbm, degi_hbm, idx_v, ones_v, z_v, acc,
                isem, ssem0, ssem1, ssem2):
    c = lax.axis_index("c")
    s = lax.axis_index("s")
    n_rows = ROWS_PER_TILE

    @pl.loop(0, 8)
    def _fill_ones(i):
        ones_v[pl.ds(i * 16, 16)] = jnp.ones((16,), jnp.float32)

    @pl.loop(0, NODE_ROWS_PER_TILE // 16)
    def _fill_zeros(i):
        z_v[pl.ds(i * 16, 16)] = jnp.zeros((16,), jnp.float32)

    pltpu.sync_copy(z_v, acc.at[pl.ds(s * NODE_ROWS_PER_TILE, NODE_ROWS_PER_TILE)])

    @pl.when(c == 0)
    def _():
        pltpu.sync_copy(src_hbm.at[pl.ds(s * n_rows, DEG_ICHUNK)], idx_v.at[0])

    @pl.when(c == 1)
    def _():
        pltpu.sync_copy(dst_hbm.at[pl.ds(s * n_rows, DEG_ICHUNK)], idx_v.at[0])

    plsc.subcore_barrier()

    def _wait_scatter(m):
        def _w(sem):
            pltpu.make_async_copy(ones_v, acc.at[idx_v.at[0, 0]], sem).wait()

        @pl.when(m == 0)
        def _():
            _w(ssem0)

        @pl.when(m == 1)
        def _():
            _w(ssem1)

        @pl.when(m == 2)
        def _():
            _w(ssem2)

    @pl.loop(0, n_rows)
    def _edges(r):
        g = r // DEG_ICHUNK
        j = r % DEG_ICHUNK
        slot = g % 3

        @pl.when(j == 0)
        def _():
            @pl.when(g > 0)
            def _():
                pltpu.make_async_copy(
                    src_hbm.at[pl.ds(0, DEG_ICHUNK)], idx_v.at[0], isem
                ).wait()

            @pl.when(g < DEG_NCHUNK - 1)
            def _():
                nslot = (g + 1) % 3
                base = s * n_rows + (g + 1) * DEG_ICHUNK

                @pl.when(c == 0)
                def _():
                    pltpu.async_copy(
                        src_hbm.at[pl.ds(base, DEG_ICHUNK)], idx_v.at[nslot], isem
                    )

                @pl.when(c == 1)
                def _():
                    pltpu.async_copy(
                        dst_hbm.at[pl.ds(base, DEG_ICHUNK)], idx_v.at[nslot], isem
                    )

        @pl.when(r >= 3)
        def _():
            _wait_scatter(r % 3)

        def _fire(sem):
            pltpu.async_copy(ones_v, acc.at[idx_v.at[slot, j]], sem, add=True)

        @pl.when(r % 3 == 0)
        def _():
            _fire(ssem0)

        @pl.when(r % 3 == 1)
        def _():
            _fire(ssem1)

        @pl.when(r % 3 == 2)
        def _():
            _fire(ssem2)

    _wait_scatter(0)
    _wait_scatter(1)
    _wait_scatter(2)
    plsc.subcore_barrier()

    pltpu.sync_copy(acc.at[pl.ds(s * NODE_ROWS_PER_TILE, NODE_ROWS_PER_TILE)], z_v)

    @pl.when(jnp.logical_and(c == 0, s == 0))
    def _():
        head = z_v[pl.ds(0, 16)]
        corr = jnp.where(
            lax.iota(jnp.int32, 16) == 0, jnp.float32(N_EXTRA), jnp.float32(0.0)
        )
        z_v[pl.ds(0, 16)] = head - corr

    @pl.when(c == 0)
    def _():
        pltpu.sync_copy(
            z_v, dego_hbm.at[pl.ds(s * NODE_ROWS_PER_TILE, NODE_ROWS_PER_TILE)]
        )

    @pl.when(c == 1)
    def _():
        pltpu.sync_copy(
            z_v, degi_hbm.at[pl.ds(s * NODE_ROWS_PER_TILE, NODE_ROWS_PER_TILE)]
        )


@functools.partial(
    pl.kernel,
    out_type=[
        jax.ShapeDtypeStruct((N_PAD, H), jnp.float32),
        jax.ShapeDtypeStruct((N_PAD, H), jnp.float32),
    ],
    mesh=_SC_MESH,
    compiler_params=pltpu.CompilerParams(use_tc_tiling_on_sc=False),
    scratch_types=[
        pltpu.VMEM((4, AGG_ICHUNK, 128), jnp.int32),
        pltpu.VMEM((4, AGG_ICHUNK, 128), jnp.int32),
        pltpu.VMEM((6, 128, H), jnp.float32),
        pltpu.VMEM((ZROWS, H), jnp.float32),
        pltpu.VMEM_SHARED((N_PAD, H), jnp.float32),
        pltpu.SemaphoreType.DMA,
        pltpu.SemaphoreType.DMA,
        pltpu.SemaphoreType.DMA,
        pltpu.SemaphoreType.DMA,
        pltpu.SemaphoreType.DMA,
        pltpu.SemaphoreType.DMA,
        pltpu.SemaphoreType.DMA,
        pltpu.SemaphoreType.DMA,
    ],
)
def _agg_kernel(ylo_hbm, yhi_hbm, src_hbm, dst_hbm, alo_hbm, ahi_hbm,
                sidx, didx, rows, z_v, acc,
                zsem, isem, gsem0, gsem1, gsem2, ssem0, ssem1, ssem2):
    c = lax.axis_index("c")
    s = lax.axis_index("s")
    n_rows = ROWS_PER_TILE

    @pl.loop(0, ZROWS)
    def _fill_zeros(i):
        @pl.loop(0, H // 16)
        def _inner(j):
            z_v[i, pl.ds(j * 16, 16)] = jnp.zeros((16,), jnp.float32)

    @pl.loop(0, NZCOPY // 8)
    def _zero_batch(kb):
        @pl.loop(0, 8)
        def _zero_fire(k):
            pltpu.async_copy(
                z_v,
                acc.at[pl.ds(s * NODE_ROWS_PER_TILE + (kb * 8 + k) * ZROWS, ZROWS)],
                zsem,
            )

        @pl.loop(0, 8)
        def _zero_drain(k):
            pltpu.make_async_copy(z_v, acc.at[pl.ds(0, ZROWS)], zsem).wait()

    pltpu.sync_copy(src_hbm.at[pl.ds(s * n_rows, AGG_ICHUNK)], sidx.at[0])
    pltpu.sync_copy(dst_hbm.at[pl.ds(s * n_rows, AGG_ICHUNK)], didx.at[0])
    pltpu.async_copy(
        src_hbm.at[pl.ds(s * n_rows + AGG_ICHUNK, AGG_ICHUNK)], sidx.at[1], isem
    )
    pltpu.async_copy(
        dst_hbm.at[pl.ds(s * n_rows + AGG_ICHUNK, AGG_ICHUNK)], didx.at[1], isem
    )

    plsc.subcore_barrier()

    def _gsem(m, fn):
        @pl.when(m == 0)
        def _():
            fn(gsem0)

        @pl.when(m == 1)
        def _():
            fn(gsem1)

        @pl.when(m == 2)
        def _():
            fn(gsem2)

    def _ssem(m, fn):
        @pl.when(m == 0)
        def _():
            fn(ssem0)

        @pl.when(m == 1)
        def _():
            fn(ssem1)

        @pl.when(m == 2)
        def _():
            fn(ssem2)

    def _fire_gather(r):
        slot = (r // AGG_ICHUNK) % 4
        j = r % AGG_ICHUNK
        buf = r % 6

        def _do(sem):
            @pl.when(c == 0)
            def _():
                pltpu.async_copy(ylo_hbm.at[sidx.at[slot, j]], rows.at[buf], sem)

            @pl.when(c == 1)
            def _():
                pltpu.async_copy(yhi_hbm.at[sidx.at[slot, j]], rows.at[buf], sem)

        _gsem(r % 3, _do)

    def _wait_gather(r):
        def _do(sem):
            pltpu.make_async_copy(ylo_hbm.at[sidx.at[0, 0]], rows.at[0], sem).wait()

        _gsem(r % 3, _do)

    def _fire_scatter(r):
        slot = (r // AGG_ICHUNK) % 4
        j = r % AGG_ICHUNK
        buf = r % 6

        def _do(sem):
            pltpu.async_copy(rows.at[buf], acc.at[didx.at[slot, j]], sem, add=True)

        _ssem(r % 3, _do)

    def _wait_scatter(r):
        def _do(sem):
            pltpu.make_async_copy(rows.at[0], acc.at[didx.at[0, 0]], sem).wait()

        _ssem(r % 3, _do)

    def _wait_iload():
        pltpu.make_async_copy(src_hbm.at[pl.ds(0, AGG_ICHUNK)], sidx.at[0], isem).wait()
        pltpu.make_async_copy(dst_hbm.at[pl.ds(0, AGG_ICHUNK)], didx.at[0], isem).wait()

    _fire_gather(0)
    _fire_gather(1)
    _fire_gather(2)

    @pl.loop(0, n_rows)
    def _edges(r):
        g = r // AGG_ICHUNK
        j = r % AGG_ICHUNK

        @pl.when(j == 0)
        def _():
            @pl.when(g < AGG_NCHUNK - 1)
            def _():
                _wait_iload()

            @pl.when(g < AGG_NCHUNK - 2)
            def _():
                nslot = (g + 2) % 4
                base = s * n_rows + (g + 2) * AGG_ICHUNK
                pltpu.async_copy(src_hbm.at[pl.ds(base, AGG_ICHUNK)], sidx.at[nslot], isem)
                pltpu.async_copy(dst_hbm.at[pl.ds(base, AGG_ICHUNK)], didx.at[nslot], isem)

        _wait_gather(r)

        @pl.when(r >= 3)
        def _():
            _wait_scatter(r % 3)

        @pl.when(r < n_rows - 3)
        def _():
            _fire_gather(r + 3)

        _fire_scatter(r)

    _wait_scatter(0)
    _wait_scatter(1)
    _wait_scatter(2)
    plsc.subcore_barrier()

    @pl.loop(0, NCOCOPY)
    def _copy_out(k):
        b = k % 2
        chunk = pl.ds(s * NODE_ROWS_PER_TILE + k * CORORS, CORORS)

        @pl.when(k >= 2)
        def _():
            def _d(sem):
                pltpu.make_async_copy(
                    rows.at[0].at[pl.ds(0, CORORS)], alo_hbm.at[pl.ds(0, CORORS)], sem
                ).wait()

            @pl.when(b == 0)
            def _():
                _d(gsem0)

            @pl.when(b == 1)
            def _():
                _d(gsem1)

        pltpu.sync_copy(acc.at[chunk], rows.at[b].at[pl.ds(0, CORORS)])

        def _f(sem):
            @pl.when(c == 0)
            def _():
                pltpu.async_copy(rows.at[b].at[pl.ds(0, CORORS)], alo_hbm.at[chunk], sem)

            @pl.when(c == 1)
            def _():
                pltpu.async_copy(rows.at[b].at[pl.ds(0, CORORS)], ahi_hbm.at[chunk], sem)

        @pl.when(b == 0)
        def _():
            _f(gsem0)

        @pl.when(b == 1)
        def _():
            _f(gsem1)

    def _drain(sem):
        pltpu.make_async_copy(
            rows.at[0].at[pl.ds(0, CORORS)], alo_hbm.at[pl.ds(0, CORORS)], sem
        ).wait()

    _drain(gsem0)
    _drain(gsem1)


_RB = 5000


def _tc0_body(x_ref, w_ref, dego_ref, ylo_ref, yhi_ref):
    so = lax.rsqrt(jnp.maximum(dego_ref[...], 1.0))
    y = jnp.dot(x_ref[...], w_ref[...], preferred_element_type=jnp.float32) * so
    ylo_ref[...] = y[:, :H]
    yhi_ref[...] = y[:, H:]


def _tc1_body(alo_ref, ahi_ref, degi_ref, dego_ref, b_ref, w_ref,
              ylo_ref, yhi_ref):
    si = lax.rsqrt(jnp.maximum(degi_ref[...], 1.0))
    so = lax.rsqrt(jnp.maximum(dego_ref[...], 1.0))
    h_lo = jnp.maximum(alo_ref[...] * si + b_ref[:, :H], 0.0)
    h_hi = jnp.maximum(ahi_ref[...] * si + b_ref[:, H:], 0.0)
    y = (
        jnp.dot(h_lo, w_ref[:H, :], preferred_element_type=jnp.float32)
        + jnp.dot(h_hi, w_ref[H:, :], preferred_element_type=jnp.float32)
    ) * so
    ylo_ref[...] = y[:, :H]
    yhi_ref[...] = y[:, H:]


def _tc2_body(alo_ref, ahi_ref, degi_ref, b_ref, out_ref):
    si = lax.rsqrt(jnp.maximum(degi_ref[...], 1.0))
    out_ref[:, :H] = alo_ref[...] * si + b_ref[:, :H]
    out_ref[:, H:] = ahi_ref[...] * si + b_ref[:, H:]


def _row_spec(cols):
    return pl.BlockSpec((_RB, cols), lambda i: (i, 0))


def _full_spec(r, cols):
    return pl.BlockSpec((r, cols), lambda i: (0, 0))


_tc0 = pl.pallas_call(
    _tc0_body,
    grid=(10,),
    in_specs=[_row_spec(D), _full_spec(D, D), _row_spec(1)],
    out_specs=[_row_spec(H), _row_spec(H)],
    out_shape=[
        jax.ShapeDtypeStruct((N, H), jnp.float32),
        jax.ShapeDtypeStruct((N, H), jnp.float32),
    ],
)

_tc1 = pl.pallas_call(
    _tc1_body,
    grid=(10,),
    in_specs=[
        _row_spec(H), _row_spec(H), _row_spec(1), _row_spec(1),
        _full_spec(1, D), _full_spec(D, D),
    ],
    out_specs=[_row_spec(H), _row_spec(H)],
    out_shape=[
        jax.ShapeDtypeStruct((N, H), jnp.float32),
        jax.ShapeDtypeStruct((N, H), jnp.float32),
    ],
)

_tc2 = pl.pallas_call(
    _tc2_body,
    grid=(10,),
    in_specs=[_row_spec(H), _row_spec(H), _row_spec(1), _full_spec(1, D)],
    out_specs=pl.BlockSpec((_RB, D), lambda i: (i, 0)),
    out_shape=jax.ShapeDtypeStruct((N, D), jnp.float32),
)


def kernel(edge_index, node_embeddings, W1, b1, W2, b2):
    src = edge_index[0].astype(jnp.int32)
    dst = edge_index[1].astype(jnp.int32)
    srcp = jnp.concatenate(
        [src, jnp.zeros((N_EXTRA,), jnp.int32)]
    ).reshape(IDX_ROWS, 128)
    dstp = jnp.concatenate(
        [dst, jnp.full((N_EXTRA,), N, jnp.int32)]
    ).reshape(IDX_ROWS, 128)

    dego, degi = _deg_kernel(srcp, dstp)
    dego = dego.reshape(N_PAD, 1)
    degi = degi.reshape(N_PAD, 1)

    b1r = b1.reshape(1, D)
    b2r = b2.reshape(1, D)

    ylo, yhi = _tc0(node_embeddings, W1, dego)
    alo, ahi = _agg_kernel(ylo, yhi, srcp, dstp)
    y2lo, y2hi = _tc1(alo, ahi, degi, dego, b1r, W2)
    a2lo, a2hi = _agg_kernel(y2lo, y2hi, srcp, dstp)
    out = _tc2(a2lo, a2hi, degi, b2r)
    return out


# speedup vs baseline: 10.7266x; 1.1681x over previous
import functools

import jax
import jax.numpy as jnp
from jax import lax
from jax.experimental import pallas as pl
from jax.experimental.pallas import tpu as pltpu
from jax.experimental.pallas import tpu_sc as plsc

N = 50000
E = 800000
D = 64
H = D // 2

N_PAD = 50176
E_PAD = 802816
N_EXTRA = E_PAD - E
IDX_ROWS = E_PAD // 128
ROWS_PER_TILE = IDX_ROWS // 16
NODE_ROWS_PER_TILE = N_PAD // 16

AGG_ICHUNK = 4
AGG_NCHUNK = ROWS_PER_TILE // AGG_ICHUNK
ZROWS = 28
NZCOPY = NODE_ROWS_PER_TILE // ZROWS
CORORS = 112
NCOCOPY = NODE_ROWS_PER_TILE // CORORS

DEG_ICHUNK = 8
DEG_NCHUNK = ROWS_PER_TILE // DEG_ICHUNK

_SC_MESH = plsc.VectorSubcoreMesh(core_axis_name="c", subcore_axis_name="s")


@functools.partial(
    pl.kernel,
    out_type=[
        jax.ShapeDtypeStruct((N_PAD,), jnp.float32),
        jax.ShapeDtypeStruct((N_PAD,), jnp.float32),
    ],
    mesh=_SC_MESH,
    compiler_params=pltpu.CompilerParams(use_tc_tiling_on_sc=False),
    scratch_types=[
        pltpu.VMEM((3, DEG_ICHUNK, 128), jnp.int32),
        pltpu.VMEM((128,), jnp.float32),
        pltpu.VMEM((NODE_ROWS_PER_TILE,), jnp.float32),
        pltpu.VMEM_SHARED((N_PAD,), jnp.float32),
        pltpu.SemaphoreType.DMA,
        pltpu.SemaphoreType.DMA,
        pltpu.SemaphoreType.DMA,
        pltpu.SemaphoreType.DMA,
    ],
)
def _deg_kernel(src_hbm, dst_hbm, dego_hbm, degi_hbm, idx_v, ones_v, z_v, acc,
                isem, ssem0, ssem1, ssem2):
    c = lax.axis_index("c")
    s = lax.axis_index("s")
    n_rows = ROWS_PER_TILE

    @pl.loop(0, 8)
    def _fill_ones(i):
        ones_v[pl.ds(i * 16, 16)] = jnp.ones((16,), jnp.float32)

    @pl.loop(0, NODE_ROWS_PER_TILE // 16)
    def _fill_zeros(i):
        z_v[pl.ds(i * 16, 16)] = jnp.zeros((16,), jnp.float32)

    pltpu.sync_copy(z_v, acc.at[pl.ds(s * NODE_ROWS_PER_TILE, NODE_ROWS_PER_TILE)])

    @pl.when(c == 0)
    def _():
        pltpu.sync_copy(src_hbm.at[pl.ds(s * n_rows, DEG_ICHUNK)], idx_v.at[0])

    @pl.when(c == 1)
    def _():
        pltpu.sync_copy(dst_hbm.at[pl.ds(s * n_rows, DEG_ICHUNK)], idx_v.at[0])

    plsc.subcore_barrier()

    def _wait_scatter(m):
        def _w(sem):
            pltpu.make_async_copy(ones_v, acc.at[idx_v.at[0, 0]], sem).wait()

        @pl.when(m == 0)
        def _():
            _w(ssem0)

        @pl.when(m == 1)
        def _():
            _w(ssem1)

        @pl.when(m == 2)
        def _():
            _w(ssem2)

    @pl.loop(0, n_rows)
    def _edges(r):
        g = r // DEG_ICHUNK
        j = r % DEG_ICHUNK
        slot = g % 3

        @pl.when(j == 0)
        def _():
            @pl.when(g > 0)
            def _():
                pltpu.make_async_copy(
                    src_hbm.at[pl.ds(0, DEG_ICHUNK)], idx_v.at[0], isem
                ).wait()

            @pl.when(g < DEG_NCHUNK - 1)
            def _():
                nslot = (g + 1) % 3
                base = s * n_rows + (g + 1) * DEG_ICHUNK

                @pl.when(c == 0)
                def _():
                    pltpu.async_copy(
                        src_hbm.at[pl.ds(base, DEG_ICHUNK)], idx_v.at[nslot], isem
                    )

                @pl.when(c == 1)
                def _():
                    pltpu.async_copy(
                        dst_hbm.at[pl.ds(base, DEG_ICHUNK)], idx_v.at[nslot], isem
                    )

        @pl.when(r >= 3)
        def _():
            _wait_scatter(r % 3)

        def _fire(sem):
            pltpu.async_copy(ones_v, acc.at[idx_v.at[slot, j]], sem, add=True)

        @pl.when(r % 3 == 0)
        def _():
            _fire(ssem0)

        @pl.when(r % 3 == 1)
        def _():
            _fire(ssem1)

        @pl.when(r % 3 == 2)
        def _():
            _fire(ssem2)

    _wait_scatter(0)
    _wait_scatter(1)
    _wait_scatter(2)
    plsc.subcore_barrier()

    pltpu.sync_copy(acc.at[pl.ds(s * NODE_ROWS_PER_TILE, NODE_ROWS_PER_TILE)], z_v)

    @pl.when(jnp.logical_and(c == 0, s == 0))
    def _():
        head = z_v[pl.ds(0, 16)]
        corr = jnp.where(
            lax.iota(jnp.int32, 16) == 0, jnp.float32(N_EXTRA), jnp.float32(0.0)
        )
        z_v[pl.ds(0, 16)] = head - corr

    @pl.when(c == 0)
    def _():
        pltpu.sync_copy(
            z_v, dego_hbm.at[pl.ds(s * NODE_ROWS_PER_TILE, NODE_ROWS_PER_TILE)]
        )

    @pl.when(c == 1)
    def _():
        pltpu.sync_copy(
            z_v, degi_hbm.at[pl.ds(s * NODE_ROWS_PER_TILE, NODE_ROWS_PER_TILE)]
        )


@functools.partial(
    pl.kernel,
    out_type=[
        jax.ShapeDtypeStruct((N_PAD, H), jnp.float32),
        jax.ShapeDtypeStruct((N_PAD, H), jnp.float32),
        jax.ShapeDtypeStruct((N_PAD, H), jnp.float32),
        jax.ShapeDtypeStruct((N_PAD, H), jnp.float32),
    ],
    mesh=_SC_MESH,
    compiler_params=pltpu.CompilerParams(
        use_tc_tiling_on_sc=False, needs_layout_passes=False
    ),
    scratch_types=[
        pltpu.VMEM((4, AGG_ICHUNK, 128), jnp.int32),
        pltpu.VMEM((4, AGG_ICHUNK, 128), jnp.int32),
        pltpu.VMEM((6, 128, H), jnp.float32),
        pltpu.VMEM((ZROWS, H), jnp.float32),
        pltpu.VMEM((CORORS,), jnp.float32),
        pltpu.VMEM((CORORS,), jnp.float32),
        pltpu.VMEM((CORORS,), jnp.float32),
        pltpu.VMEM((CORORS,), jnp.float32),
        pltpu.VMEM((1, H), jnp.float32),
        pltpu.VMEM_SHARED((N_PAD, H), jnp.float32),
        pltpu.SemaphoreType.DMA,
        pltpu.SemaphoreType.DMA,
        pltpu.SemaphoreType.DMA,
        pltpu.SemaphoreType.DMA,
        pltpu.SemaphoreType.DMA,
        pltpu.SemaphoreType.DMA,
        pltpu.SemaphoreType.DMA,
        pltpu.SemaphoreType.DMA,
    ],
)
def _mega_kernel(ylo_hbm, yhi_hbm, src_hbm, dst_hbm, dego_hbm, degi_hbm, b1_hbm,
                 y2lo_hbm, y2hi_hbm, alo_hbm, ahi_hbm,
                 sidx, didx, rows, z_v, dob, dib, sob, sib, bbuf, acc,
                 zsem, isem, gsem0, gsem1, gsem2, ssem0, ssem1, ssem2):
    c = lax.axis_index("c")
    s = lax.axis_index("s")
    n_rows = ROWS_PER_TILE

    @pl.loop(0, ZROWS)
    def _fill_zeros(i):
        @pl.loop(0, H // 16)
        def _inner(j):
            z_v[i, pl.ds(j * 16, 16)] = jnp.zeros((16,), jnp.float32)

    @pl.loop(0, NZCOPY // 8)
    def _zero_batch(kb):
        @pl.loop(0, 8)
        def _zero_fire(k):
            pltpu.async_copy(
                z_v,
                acc.at[pl.ds(s * NODE_ROWS_PER_TILE + (kb * 8 + k) * ZROWS, ZROWS)],
                zsem,
            )

        @pl.loop(0, 8)
        def _zero_drain(k):
            pltpu.make_async_copy(z_v, acc.at[pl.ds(0, ZROWS)], zsem).wait()

    def _prime_idx():
        pltpu.sync_copy(src_hbm.at[pl.ds(s * n_rows, AGG_ICHUNK)], sidx.at[0])
        pltpu.sync_copy(dst_hbm.at[pl.ds(s * n_rows, AGG_ICHUNK)], didx.at[0])
        pltpu.async_copy(
            src_hbm.at[pl.ds(s * n_rows + AGG_ICHUNK, AGG_ICHUNK)], sidx.at[1], isem
        )
        pltpu.async_copy(
            dst_hbm.at[pl.ds(s * n_rows + AGG_ICHUNK, AGG_ICHUNK)], didx.at[1], isem
        )

    _prime_idx()
    plsc.subcore_barrier()

    def _gsem(m, fn):
        @pl.when(m == 0)
        def _():
            fn(gsem0)

        @pl.when(m == 1)
        def _():
            fn(gsem1)

        @pl.when(m == 2)
        def _():
            fn(gsem2)

    def _ssem(m, fn):
        @pl.when(m == 0)
        def _():
            fn(ssem0)

        @pl.when(m == 1)
        def _():
            fn(ssem1)

        @pl.when(m == 2)
        def _():
            fn(ssem2)

    def _wait_iload():
        pltpu.make_async_copy(src_hbm.at[pl.ds(0, AGG_ICHUNK)], sidx.at[0], isem).wait()
        pltpu.make_async_copy(dst_hbm.at[pl.ds(0, AGG_ICHUNK)], didx.at[0], isem).wait()

    def _agg_pass(tlo, thi):
        def _fire_gather(r):
            slot = (r // AGG_ICHUNK) % 4
            j = r % AGG_ICHUNK
            buf = r % 6

            def _do(sem):
                @pl.when(c == 0)
                def _():
                    pltpu.async_copy(tlo.at[sidx.at[slot, j]], rows.at[buf], sem)

                @pl.when(c == 1)
                def _():
                    pltpu.async_copy(thi.at[sidx.at[slot, j]], rows.at[buf], sem)

            _gsem(r % 3, _do)

        def _wait_gather(r):
            def _do(sem):
                pltpu.make_async_copy(tlo.at[sidx.at[0, 0]], rows.at[0], sem).wait()

            _gsem(r % 3, _do)

        def _fire_scatter(r):
            slot = (r // AGG_ICHUNK) % 4
            j = r % AGG_ICHUNK
            buf = r % 6

            def _do(sem):
                pltpu.async_copy(rows.at[buf], acc.at[didx.at[slot, j]], sem, add=True)

            _ssem(r % 3, _do)

        def _wait_scatter(r):
            def _do(sem):
                pltpu.make_async_copy(rows.at[0], acc.at[didx.at[0, 0]], sem).wait()

            _ssem(r % 3, _do)

        _fire_gather(0)
        _fire_gather(1)
        _fire_gather(2)

        @pl.loop(0, n_rows)
        def _edges(r):
            g = r // AGG_ICHUNK
            j = r % AGG_ICHUNK

            @pl.when(j == 0)
            def _():
                @pl.when(g < AGG_NCHUNK - 1)
                def _():
                    _wait_iload()

                @pl.when(g < AGG_NCHUNK - 2)
                def _():
                    nslot = (g + 2) % 4
                    base = s * n_rows + (g + 2) * AGG_ICHUNK
                    pltpu.async_copy(src_hbm.at[pl.ds(base, AGG_ICHUNK)], sidx.at[nslot], isem)
                    pltpu.async_copy(dst_hbm.at[pl.ds(base, AGG_ICHUNK)], didx.at[nslot], isem)

            _wait_gather(r)

            @pl.when(r >= 3)
            def _():
                _wait_scatter(r % 3)

            @pl.when(r < n_rows - 3)
            def _():
                _fire_gather(r + 3)

            _fire_scatter(r)

        _wait_scatter(0)
        _wait_scatter(1)
        _wait_scatter(2)

    _agg_pass(ylo_hbm, yhi_hbm)
    plsc.subcore_barrier()

    def _vrsqrt(x):
        i = plsc.bitcast(x, jnp.int32)
        y = plsc.bitcast(jnp.int32(0x5F3759DF) - (i >> 1), jnp.float32)
        for _ in range(3):
            y = y * (1.5 - 0.5 * x * y * y)
        return y

    pltpu.sync_copy(b1_hbm.at[pl.ds(c, 1)], bbuf)
    b_a = bbuf[0, pl.ds(0, 16)]
    b_b = bbuf[0, pl.ds(16, 16)]

    @pl.loop(0, NCOCOPY)
    def _phase_b(k):
        b = k % 2
        row0 = s * NODE_ROWS_PER_TILE + k * CORORS
        chunk = pl.ds(row0, CORORS)

        @pl.when(k >= 2)
        def _():
            def _d(sem):
                pltpu.make_async_copy(
                    rows.at[0].at[pl.ds(0, CORORS)], y2lo_hbm.at[pl.ds(0, CORORS)], sem
                ).wait()

            @pl.when(b == 0)
            def _():
                _d(gsem0)

            @pl.when(b == 1)
            def _():
                _d(gsem1)

        pltpu.sync_copy(acc.at[chunk], rows.at[b].at[pl.ds(0, CORORS)])
        pltpu.sync_copy(dego_hbm.at[chunk], dob)
        pltpu.sync_copy(degi_hbm.at[chunk], dib)

        @pl.loop(0, CORORS // 16)
        def _rsq(t):
            sl = pl.ds(t * 16, 16)
            sob[sl] = _vrsqrt(jnp.maximum(dob[sl], 1.0))
            sib[sl] = _vrsqrt(jnp.maximum(dib[sl], 1.0))

        @pl.loop(0, CORORS // 16)
        def _rowgrp(t):
            sivec = sib[pl.ds(t * 16, 16)]
            sovec = sob[pl.ds(t * 16, 16)]
            for q in range(16):
                i = t * 16 + q
                si = sivec[q]
                so = sovec[q]
                va = rows[b, i, pl.ds(0, 16)]
                vb = rows[b, i, pl.ds(16, 16)]
                rows[b, i, pl.ds(0, 16)] = jnp.maximum(va * si + b_a, 0.0) * so
                rows[b, i, pl.ds(16, 16)] = jnp.maximum(vb * si + b_b, 0.0) * so

        def _f(sem):
            @pl.when(c == 0)
            def _():
                pltpu.async_copy(rows.at[b].at[pl.ds(0, CORORS)], y2lo_hbm.at[chunk], sem)

            @pl.when(c == 1)
            def _():
                pltpu.async_copy(rows.at[b].at[pl.ds(0, CORORS)], y2hi_hbm.at[chunk], sem)

        @pl.when(b == 0)
        def _():
            _f(gsem0)

        @pl.when(b == 1)
        def _():
            _f(gsem1)

        @pl.when(k >= 1)
        def _():
            @pl.loop(0, 4)
            def _zw(q):
                pltpu.make_async_copy(z_v, acc.at[pl.ds(0, ZROWS)], zsem).wait()

        @pl.loop(0, 4)
        def _zf(q):
            pltpu.async_copy(z_v, acc.at[pl.ds(row0 + q * ZROWS, ZROWS)], zsem)

    def _drain_y2(sem):
        pltpu.make_async_copy(
            rows.at[0].at[pl.ds(0, CORORS)], y2lo_hbm.at[pl.ds(0, CORORS)], sem
        ).wait()

    _drain_y2(gsem0)
    _drain_y2(gsem1)

    @pl.loop(0, 4)
    def _zw_last(q):
        pltpu.make_async_copy(z_v, acc.at[pl.ds(0, ZROWS)], zsem).wait()

    _prime_idx()
    plsc.subcore_barrier()

    _agg_pass(y2lo_hbm, y2hi_hbm)
    plsc.subcore_barrier()

    @pl.loop(0, NCOCOPY)
    def _copy_out(k):
        b = k % 2
        chunk = pl.ds(s * NODE_ROWS_PER_TILE + k * CORORS, CORORS)

        @pl.when(k >= 2)
        def _():
            def _d(sem):
                pltpu.make_async_copy(
                    rows.at[0].at[pl.ds(0, CORORS)], alo_hbm.at[pl.ds(0, CORORS)], sem
                ).wait()

            @pl.when(b == 0)
            def _():
                _d(gsem0)

            @pl.when(b == 1)
            def _():
                _d(gsem1)

        pltpu.sync_copy(acc.at[chunk], rows.at[b].at[pl.ds(0, CORORS)])

        def _f(sem):
            @pl.when(c == 0)
            def _():
                pltpu.async_copy(rows.at[b].at[pl.ds(0, CORORS)], alo_hbm.at[chunk], sem)

            @pl.when(c == 1)
            def _():
                pltpu.async_copy(rows.at[b].at[pl.ds(0, CORORS)], ahi_hbm.at[chunk], sem)

        @pl.when(b == 0)
        def _():
            _f(gsem0)

        @pl.when(b == 1)
        def _():
            _f(gsem1)

    def _drain(sem):
        pltpu.make_async_copy(
            rows.at[0].at[pl.ds(0, CORORS)], alo_hbm.at[pl.ds(0, CORORS)], sem
        ).wait()

    _drain(gsem0)
    _drain(gsem1)


_RB = N_PAD // 8


def _tc0_body(x_ref, w_ref, dego_ref, ylo_ref, yhi_ref):
    i = pl.program_id(0)
    so = lax.rsqrt(jnp.maximum(dego_ref[pl.ds(i * _RB, _RB)], 1.0))[:, None]
    y = jnp.dot(x_ref[...], w_ref[...], preferred_element_type=jnp.float32) * so
    ylo_ref[...] = y[:, :H]
    yhi_ref[...] = y[:, H:]


def _tc2_body(alo_ref, ahi_ref, degi_ref, b_ref, w_ref, out_ref):
    i = pl.program_id(0)
    si = lax.rsqrt(jnp.maximum(degi_ref[pl.ds(i * _RB, _RB)], 1.0))[:, None]
    out_ref[...] = (
        jnp.dot(alo_ref[...] * si, w_ref[:H, :], preferred_element_type=jnp.float32)
        + jnp.dot(ahi_ref[...] * si, w_ref[H:, :], preferred_element_type=jnp.float32)
        + b_ref[...]
    )


def _row_spec(cols):
    return pl.BlockSpec((_RB, cols), lambda i: (i, 0))


def _vec_spec():
    return pl.BlockSpec((N_PAD,), lambda i: (0,))


def _full_spec(r, cols):
    return pl.BlockSpec((r, cols), lambda i: (0, 0))


_tc0 = pl.pallas_call(
    _tc0_body,
    grid=(8,),
    in_specs=[_row_spec(D), _full_spec(D, D), _vec_spec()],
    out_specs=[_row_spec(H), _row_spec(H)],
    out_shape=[
        jax.ShapeDtypeStruct((N_PAD, H), jnp.float32),
        jax.ShapeDtypeStruct((N_PAD, H), jnp.float32),
    ],
)

_tc2 = pl.pallas_call(
    _tc2_body,
    grid=(8,),
    in_specs=[
        _row_spec(H), _row_spec(H), _vec_spec(), _full_spec(1, D),
        _full_spec(D, D),
    ],
    out_specs=pl.BlockSpec((_RB, D), lambda i: (i, 0)),
    out_shape=jax.ShapeDtypeStruct((N, D), jnp.float32),
)


def kernel(edge_index, node_embeddings, W1, b1, W2, b2):
    src = edge_index[0].astype(jnp.int32)
    dst = edge_index[1].astype(jnp.int32)
    srcp = jnp.concatenate(
        [src, jnp.zeros((N_EXTRA,), jnp.int32)]
    ).reshape(IDX_ROWS, 128)
    dstp = jnp.concatenate(
        [dst, jnp.full((N_EXTRA,), N, jnp.int32)]
    ).reshape(IDX_ROWS, 128)

    dego, degi = _deg_kernel(srcp, dstp)

    ylo, yhi = _tc0(node_embeddings, W1, dego)
    _y2lo, _y2hi, a2lo, a2hi = _mega_kernel(
        ylo, yhi, srcp, dstp, dego, degi, b1.reshape(2, H)
    )
    out = _tc2(a2lo, a2hi, degi, b2.reshape(1, D), W2)
    return out
